# Initial kernel scaffold; baseline (speedup 1.0000x reference)
import jax
import jax.numpy as jnp
from jax.experimental import pallas as pl


def kernel(patch, patch_lib):
    raise NotImplementedError("write your pallas kernel here")


# speedup vs baseline: 1.2258x; 1.2258x over previous
import functools

import jax
import jax.numpy as jnp
from jax.experimental import pallas as pl
from jax.experimental.pallas import tpu as pltpu

Q = 1024
D = 32
W = 512
LANES = 128
TOPK = 80


def _nn_kernel(patch_ref, qsq_ref, lib_ref, ksq_ref,
               smap_ref, idx_ref, s_ref,
               run_val, run_blk, *, nsteps):
    step = pl.program_id(0)

    @pl.when(step == 0)
    def _init():
        run_val[:, :] = jnp.full((Q, LANES), 1e37, dtype=jnp.float32)
        run_blk[:, :] = jnp.zeros((Q, LANES), dtype=jnp.int32)

    r = jax.lax.dot_general(
        patch_ref[:, :], lib_ref[:, :],
        (((1,), (0,)), ((), ())),
        preferred_element_type=jnp.float32)
    t = qsq_ref[:, :] + ksq_ref[:, :]
    d2 = t - (r + r)

    for w in range(W // LANES):
        m_w = d2[:, w * LANES:(w + 1) * LANES]
        cg = step * (W // LANES) + w
        pred = m_w < run_val[:, :]
        run_val[:, :] = jnp.where(pred, m_w, run_val[:, :])
        run_blk[:, :] = jnp.where(pred, jnp.full((Q, LANES), cg, jnp.int32),
                                  run_blk[:, :])

    @pl.when(step == nsteps - 1)
    def _fini():
        rv = run_val[:, :]
        m1 = jnp.min(rv, axis=1, keepdims=True)
        lane = jax.lax.broadcasted_iota(jnp.int32, (Q, LANES), 1)
        gkey = run_blk[:, :] * LANES + lane
        cand = jnp.where(rv == m1, gkey, jnp.int32(2**31 - 1))
        idx_ref[:, :] = jnp.min(cand, axis=1, keepdims=True)

        dist = jnp.sqrt(jnp.maximum(m1, 1e-12))
        smap_ref[:, :] = dist

        def body(_, carry):
            remaining, total, taken = carry
            m = jnp.max(remaining, axis=(0, 1), keepdims=True)
            cnt = jnp.sum((remaining == m).astype(jnp.float32),
                          axis=(0, 1), keepdims=True)
            take = jnp.minimum(cnt, jnp.float32(TOPK) - taken)
            total = total + m * take
            taken = taken + take
            remaining = jnp.where(remaining == m,
                                  jnp.float32(-1.0), remaining)
            return remaining, total, taken

        zero = jnp.zeros((1, 1), jnp.float32)
        _, total, _ = jax.lax.fori_loop(0, TOPK, body, (dist, zero, zero))
        s_ref[:, :] = total / jnp.float32(TOPK)


def kernel(patch, patch_lib):
    k = patch_lib.shape[0]
    kp = pl.cdiv(k, W) * W
    nsteps = kp // W

    lib_t = jnp.pad(patch_lib.T, ((0, 0), (0, kp - k)))
    qsq = jnp.sum(patch * patch, axis=1, keepdims=True)
    ksq = jnp.pad(jnp.sum(patch_lib * patch_lib, axis=1)[None, :],
                  ((0, 0), (0, kp - k)), constant_values=1e30)

    smap_col, idx_col, s11 = pl.pallas_call(
        functools.partial(_nn_kernel, nsteps=nsteps),
        grid=(nsteps,),
        in_specs=[
            pl.BlockSpec((Q, D), lambda i: (0, 0)),
            pl.BlockSpec((Q, 1), lambda i: (0, 0)),
            pl.BlockSpec((D, W), lambda i: (0, i)),
            pl.BlockSpec((1, W), lambda i: (0, i)),
        ],
        out_specs=[
            pl.BlockSpec((Q, 1), lambda i: (0, 0)),
            pl.BlockSpec((Q, 1), lambda i: (0, 0)),
            pl.BlockSpec((1, 1), lambda i: (0, 0)),
        ],
        out_shape=[
            jax.ShapeDtypeStruct((Q, 1), jnp.float32),
            jax.ShapeDtypeStruct((Q, 1), jnp.int32),
            jax.ShapeDtypeStruct((1, 1), jnp.float32),
        ],
        scratch_shapes=[
            pltpu.VMEM((Q, LANES), jnp.float32),
            pltpu.VMEM((Q, LANES), jnp.int32),
        ],
    )(patch, qsq, lib_t, ksq)

    s_map = smap_col.reshape(1, 1, Q)
    min_idx = idx_col.reshape(Q)
    s = s11.reshape(())
    return (s_map, min_idx, s)
